# baseline (speedup 1.0000x reference)
import functools

import jax
import jax.numpy as jnp
from jax.experimental import pallas as pl


_COMMIT = 0.25
_KT = 512
_BIG = 1e9


def _vq_body(x_ref, win_ref, bin_ref, cb_ref, c2_ref, wout_ref, bout_ref,
             out_ref, idx_ref, loss_ref, *, n_b, commit_scale):
    b = pl.program_id(0)
    T = x_ref.shape[2]
    CD = cb_ref.shape[1]
    K = cb_ref.shape[0]

    xb = x_ref[0]
    zm2 = jnp.dot(win_ref[...], xb, preferred_element_type=jnp.float32)
    zm2 = zm2 + bin_ref[...]
    ztm2 = zm2.T
    zt = ztm2 * -0.5

    iota_f = jax.lax.broadcasted_iota(
        jnp.int32, (T, _KT), 1).astype(jnp.float32)

    run_min = jnp.full((T, 1), _BIG, dtype=jnp.float32)
    run_idx = jnp.zeros((T, 1), dtype=jnp.float32)
    run_zq = jnp.zeros((T, CD), dtype=jnp.float32)

    dn = (((1,), (1,)), ((), ()))

    for j in range(K // _KT):
        lo = j * _KT
        cb_tile = cb_ref[lo:lo + _KT, :]
        scores = jax.lax.dot_general(
            ztm2, cb_tile, dn,
            preferred_element_type=jnp.float32)
        dist = scores + c2_ref[:, lo:lo + _KT]
        tmin = jnp.min(dist, axis=1, keepdims=True)
        upd = tmin < run_min
        nm = jnp.where(upd, tmin, run_min)
        key = jnp.where(dist == nm, iota_f, _BIG)
        larg = jnp.min(key, axis=1, keepdims=True)
        onehot = (iota_f == larg).astype(jnp.float32)
        tzq = jnp.dot(onehot, cb_tile,
                      preferred_element_type=jnp.float32)
        run_min = nm
        run_idx = jnp.where(upd, larg + jnp.float32(lo), run_idx)
        run_zq = jnp.where(upd, tzq, run_zq)

    outb = jnp.dot(wout_ref[...], run_zq.T,
                   preferred_element_type=jnp.float32) + bout_ref[...]
    out_ref[0] = outb
    idx_ref[...] = run_idx.astype(jnp.int32)

    diff = run_zq - zt
    part = jnp.sum(diff * diff, axis=(0, 1), keepdims=True)

    @pl.when(b == 0)
    def _init():
        loss_ref[...] = part

    @pl.when(b > 0)
    def _acc():
        loss_ref[...] += part

    @pl.when(b == n_b - 1)
    def _scale():
        loss_ref[...] = loss_ref[...] * commit_scale


@jax.jit
def kernel(x, W_in, b_in, codebook, W_out, b_out):
    B, D, T = x.shape
    K, CD = codebook.shape
    M = B * T

    W_in_m2 = W_in * -2.0
    bin2 = (b_in * -2.0).reshape(CD, 1)
    c2row = jnp.sum(codebook * codebook, axis=1)[None, :]
    bout2 = b_out.reshape(D, 1)

    TS = 2
    TB = T // TS
    n_steps = B * TS
    body = functools.partial(_vq_body, n_b=n_steps,
                             commit_scale=_COMMIT / (M * CD))

    out, idx, loss = pl.pallas_call(
        body,
        grid=(n_steps,),
        in_specs=[
            pl.BlockSpec((1, D, TB), lambda s: (s // TS, 0, s % TS)),
            pl.BlockSpec((CD, D), lambda s: (0, 0)),
            pl.BlockSpec((CD, 1), lambda s: (0, 0)),
            pl.BlockSpec((K, CD), lambda s: (0, 0)),
            pl.BlockSpec((1, K), lambda s: (0, 0)),
            pl.BlockSpec((D, CD), lambda s: (0, 0)),
            pl.BlockSpec((D, 1), lambda s: (0, 0)),
        ],
        out_specs=[
            pl.BlockSpec((1, D, TB), lambda s: (s // TS, 0, s % TS)),
            pl.BlockSpec((TB, 1), lambda s: (s, 0)),
            pl.BlockSpec((1, 1), lambda s: (0, 0)),
        ],
        out_shape=[
            jax.ShapeDtypeStruct((B, D, T), jnp.float32),
            jax.ShapeDtypeStruct((M, 1), jnp.int32),
            jax.ShapeDtypeStruct((1, 1), jnp.float32),
        ],
    )(x, W_in_m2, bin2, codebook, c2row, W_out, bout2)

    q = idx.reshape(1, B, T)
    return out, q, loss.reshape(())


# speedup vs baseline: 1.0616x; 1.0616x over previous
import functools

import jax
import jax.numpy as jnp
from jax.experimental import pallas as pl


_COMMIT = 0.25
_KT = 512
_BIG = 1e9


def _vq_body(x_ref, win_ref, bin_ref, cb_ref, c2_ref, wout_ref, bout_ref,
             out_ref, idx_ref, loss_ref, *, n_b, commit_scale):
    b = pl.program_id(0)
    T = x_ref.shape[2]
    CD = cb_ref.shape[1]
    K = cb_ref.shape[0]

    xb = x_ref[0]
    zm2 = jnp.dot(win_ref[...], xb, preferred_element_type=jnp.float32)
    zm2 = zm2 + bin_ref[...]
    ztm2 = zm2.T
    zt = ztm2 * -0.5

    iota_f = jax.lax.broadcasted_iota(
        jnp.int32, (T, _KT), 1).astype(jnp.float32)

    run_min = jnp.full((T, 1), _BIG, dtype=jnp.float32)
    run_idx = jnp.zeros((T, 1), dtype=jnp.float32)
    run_zq = jnp.zeros((T, CD), dtype=jnp.float32)

    dn = (((1,), (1,)), ((), ()))

    for j in range(K // _KT):
        lo = j * _KT
        cb_tile = cb_ref[lo:lo + _KT, :]
        scores = jax.lax.dot_general(
            ztm2, cb_tile, dn,
            preferred_element_type=jnp.float32)
        dist = scores + c2_ref[:, lo:lo + _KT]
        tmin = jnp.min(dist, axis=1, keepdims=True)
        upd = tmin < run_min
        nm = jnp.where(upd, tmin, run_min)
        key = jnp.where(dist == nm, iota_f, _BIG)
        larg = jnp.min(key, axis=1, keepdims=True)
        onehot = (iota_f == larg).astype(jnp.bfloat16)
        tzq = jnp.dot(onehot, cb_tile.astype(jnp.bfloat16),
                      preferred_element_type=jnp.float32)
        run_min = nm
        run_idx = jnp.where(upd, larg + jnp.float32(lo), run_idx)
        run_zq = jnp.where(upd, tzq, run_zq)

    outb = jnp.dot(wout_ref[...].astype(jnp.bfloat16),
                   run_zq.T.astype(jnp.bfloat16),
                   preferred_element_type=jnp.float32) + bout_ref[...]
    out_ref[0] = outb
    idx_ref[...] = run_idx.astype(jnp.int32)

    diff = run_zq - zt
    part = jnp.sum(diff * diff, axis=(0, 1), keepdims=True)

    @pl.when(b == 0)
    def _init():
        loss_ref[...] = part

    @pl.when(b > 0)
    def _acc():
        loss_ref[...] += part

    @pl.when(b == n_b - 1)
    def _scale():
        loss_ref[...] = loss_ref[...] * commit_scale


@jax.jit
def kernel(x, W_in, b_in, codebook, W_out, b_out):
    B, D, T = x.shape
    K, CD = codebook.shape
    M = B * T

    W_in_m2 = W_in * -2.0
    bin2 = (b_in * -2.0).reshape(CD, 1)
    c2row = jnp.sum(codebook * codebook, axis=1)[None, :]
    bout2 = b_out.reshape(D, 1)

    TS = 1
    TB = T // TS
    n_steps = B * TS
    body = functools.partial(_vq_body, n_b=n_steps,
                             commit_scale=_COMMIT / (M * CD))

    out, idx, loss = pl.pallas_call(
        body,
        grid=(n_steps,),
        in_specs=[
            pl.BlockSpec((1, D, TB), lambda s: (s // TS, 0, s % TS)),
            pl.BlockSpec((CD, D), lambda s: (0, 0)),
            pl.BlockSpec((CD, 1), lambda s: (0, 0)),
            pl.BlockSpec((K, CD), lambda s: (0, 0)),
            pl.BlockSpec((1, K), lambda s: (0, 0)),
            pl.BlockSpec((D, CD), lambda s: (0, 0)),
            pl.BlockSpec((D, 1), lambda s: (0, 0)),
        ],
        out_specs=[
            pl.BlockSpec((1, D, TB), lambda s: (s // TS, 0, s % TS)),
            pl.BlockSpec((TB, 1), lambda s: (s, 0)),
            pl.BlockSpec((1, 1), lambda s: (0, 0)),
        ],
        out_shape=[
            jax.ShapeDtypeStruct((B, D, T), jnp.float32),
            jax.ShapeDtypeStruct((M, 1), jnp.int32),
            jax.ShapeDtypeStruct((1, 1), jnp.float32),
        ],
    )(x, W_in_m2, bin2, codebook, c2row, W_out, bout2)

    q = idx.reshape(1, B, T)
    return out, q, loss.reshape(())


# speedup vs baseline: 1.1320x; 1.0663x over previous
import functools

import jax
import jax.numpy as jnp
from jax.experimental import pallas as pl


_COMMIT = 0.25
_KT = 512
_BIG = 1e9


def _vq_body(x_ref, win_ref, bin_ref, cb_ref, c2_ref, wout_ref, bout_ref,
             out_ref, idx_ref, loss_ref, *, n_b, commit_scale):
    b = pl.program_id(0)
    T = x_ref.shape[2]
    CD = cb_ref.shape[1]
    K = cb_ref.shape[0]

    xb = x_ref[0]
    zm2 = jnp.dot(win_ref[...], xb, preferred_element_type=jnp.float32)
    zm2 = zm2 + bin_ref[...]
    ztm2 = zm2.T
    zt = ztm2 * -0.5

    iota_f = jax.lax.broadcasted_iota(
        jnp.int32, (T, _KT), 1).astype(jnp.float32)

    run_min = jnp.full((T, 1), _BIG, dtype=jnp.float32)
    run_idx = jnp.zeros((T, 1), dtype=jnp.float32)
    run_zq = jnp.zeros((T, CD), dtype=jnp.float32)

    dn = (((1,), (1,)), ((), ()))

    for j in range(K // _KT):
        lo = j * _KT
        cb_tile = cb_ref[lo:lo + _KT, :]
        scores = jax.lax.dot_general(
            ztm2, cb_tile, dn,
            preferred_element_type=jnp.float32)
        dist = scores + c2_ref[:, lo:lo + _KT]
        tmin = jnp.min(dist, axis=1, keepdims=True)
        upd = tmin < run_min
        nm = jnp.where(upd, tmin, run_min)
        key = jnp.where(dist == nm, iota_f, _BIG)
        larg = jnp.min(key, axis=1, keepdims=True)
        onehot = (iota_f == larg).astype(jnp.float32)
        tzq = jnp.dot(onehot, cb_tile,
                      preferred_element_type=jnp.float32)
        run_min = nm
        run_idx = jnp.where(upd, larg + jnp.float32(lo), run_idx)
        run_zq = jnp.where(upd, tzq, run_zq)

    outb = jnp.dot(wout_ref[...].astype(jnp.bfloat16),
                   run_zq.T.astype(jnp.bfloat16),
                   preferred_element_type=jnp.float32) + bout_ref[...]
    out_ref[0] = outb
    idx_ref[...] = run_idx.astype(jnp.int32)

    diff = run_zq - zt
    part = jnp.sum(diff * diff, axis=(0, 1), keepdims=True)

    @pl.when(b == 0)
    def _init():
        loss_ref[...] = part

    @pl.when(b > 0)
    def _acc():
        loss_ref[...] += part

    @pl.when(b == n_b - 1)
    def _scale():
        loss_ref[...] = loss_ref[...] * commit_scale


@jax.jit
def kernel(x, W_in, b_in, codebook, W_out, b_out):
    B, D, T = x.shape
    K, CD = codebook.shape
    M = B * T

    W_in_m2 = W_in * -2.0
    bin2 = (b_in * -2.0).reshape(CD, 1)
    c2row = jnp.sum(codebook * codebook, axis=1)[None, :]
    bout2 = b_out.reshape(D, 1)

    TS = 1
    TB = T // TS
    n_steps = B * TS
    body = functools.partial(_vq_body, n_b=n_steps,
                             commit_scale=_COMMIT / (M * CD))

    out, idx, loss = pl.pallas_call(
        body,
        grid=(n_steps,),
        in_specs=[
            pl.BlockSpec((1, D, TB), lambda s: (s // TS, 0, s % TS)),
            pl.BlockSpec((CD, D), lambda s: (0, 0)),
            pl.BlockSpec((CD, 1), lambda s: (0, 0)),
            pl.BlockSpec((K, CD), lambda s: (0, 0)),
            pl.BlockSpec((1, K), lambda s: (0, 0)),
            pl.BlockSpec((D, CD), lambda s: (0, 0)),
            pl.BlockSpec((D, 1), lambda s: (0, 0)),
        ],
        out_specs=[
            pl.BlockSpec((1, D, TB), lambda s: (s // TS, 0, s % TS)),
            pl.BlockSpec((TB, 1), lambda s: (s, 0)),
            pl.BlockSpec((1, 1), lambda s: (0, 0)),
        ],
        out_shape=[
            jax.ShapeDtypeStruct((B, D, T), jnp.float32),
            jax.ShapeDtypeStruct((M, 1), jnp.int32),
            jax.ShapeDtypeStruct((1, 1), jnp.float32),
        ],
    )(x, W_in_m2, bin2, codebook, c2row, W_out, bout2)

    q = idx.reshape(1, B, T)
    return out, q, loss.reshape(())


# speedup vs baseline: 1.1428x; 1.0095x over previous
import functools

import jax
import jax.numpy as jnp
from jax.experimental import pallas as pl


_COMMIT = 0.25
_KT = 512
_BIG = 1e9


def _vq_body(x_ref, win_ref, bin_ref, cb_ref, c2_ref, wout_ref, bout_ref,
             out_ref, idx_ref, loss_ref, *, n_b, commit_scale):
    b = pl.program_id(0)
    T = x_ref.shape[2]
    CD = cb_ref.shape[1]
    K = cb_ref.shape[0]

    xb = x_ref[0]
    zm2 = jnp.dot(win_ref[...], xb, preferred_element_type=jnp.float32)
    zm2 = zm2 + bin_ref[...]
    ztm2 = zm2.T
    zt = ztm2 * -0.5

    iota_f = jax.lax.broadcasted_iota(
        jnp.int32, (T, _KT), 1).astype(jnp.float32)

    run_min = jnp.full((T, 1), _BIG, dtype=jnp.float32)
    run_idx = jnp.zeros((T, 1), dtype=jnp.float32)
    run_zq = jnp.zeros((T, CD), dtype=jnp.float32)

    dn = (((1,), (1,)), ((), ()))

    for j in range(K // _KT):
        lo = j * _KT
        cb_tile = cb_ref[lo:lo + _KT, :]
        scores = jax.lax.dot_general(
            ztm2, cb_tile, dn,
            preferred_element_type=jnp.float32)
        dist = scores + c2_ref[:, lo:lo + _KT]
        tmin = jnp.min(dist, axis=1, keepdims=True)
        upd = tmin < run_min
        nm = jnp.where(upd, tmin, run_min)
        key = jnp.where(dist == nm, iota_f, _BIG)
        larg = jnp.min(key, axis=1, keepdims=True)
        onehot = (iota_f == larg).astype(jnp.float32)
        tzq = jnp.dot(onehot, cb_tile,
                      preferred_element_type=jnp.float32)
        run_min = nm
        run_idx = jnp.where(upd, larg + jnp.float32(lo), run_idx)
        run_zq = jnp.where(upd, tzq, run_zq)

    outb = jnp.dot(wout_ref[...], run_zq.T,
                   preferred_element_type=jnp.float32) + bout_ref[...]
    out_ref[0] = outb
    idx_ref[...] = run_idx.astype(jnp.int32)

    diff = run_zq - zt
    part = jnp.sum(diff * diff, axis=(0, 1), keepdims=True)

    @pl.when(b == 0)
    def _init():
        loss_ref[...] = part

    @pl.when(b > 0)
    def _acc():
        loss_ref[...] += part

    @pl.when(b == n_b - 1)
    def _scale():
        loss_ref[...] = loss_ref[...] * commit_scale


@jax.jit
def kernel(x, W_in, b_in, codebook, W_out, b_out):
    B, D, T = x.shape
    K, CD = codebook.shape
    M = B * T

    W_in_m2 = W_in * -2.0
    bin2 = (b_in * -2.0).reshape(CD, 1)
    c2row = jnp.sum(codebook * codebook, axis=1)[None, :]
    bout2 = b_out.reshape(D, 1)

    TS = 1
    TB = T // TS
    n_steps = B * TS
    body = functools.partial(_vq_body, n_b=n_steps,
                             commit_scale=_COMMIT / (M * CD))

    out, idx, loss = pl.pallas_call(
        body,
        grid=(n_steps,),
        in_specs=[
            pl.BlockSpec((1, D, TB), lambda s: (s // TS, 0, s % TS)),
            pl.BlockSpec((CD, D), lambda s: (0, 0)),
            pl.BlockSpec((CD, 1), lambda s: (0, 0)),
            pl.BlockSpec((K, CD), lambda s: (0, 0)),
            pl.BlockSpec((1, K), lambda s: (0, 0)),
            pl.BlockSpec((D, CD), lambda s: (0, 0)),
            pl.BlockSpec((D, 1), lambda s: (0, 0)),
        ],
        out_specs=[
            pl.BlockSpec((1, D, TB), lambda s: (s // TS, 0, s % TS)),
            pl.BlockSpec((TB, 1), lambda s: (s, 0)),
            pl.BlockSpec((1, 1), lambda s: (0, 0)),
        ],
        out_shape=[
            jax.ShapeDtypeStruct((B, D, T), jnp.float32),
            jax.ShapeDtypeStruct((M, 1), jnp.int32),
            jax.ShapeDtypeStruct((1, 1), jnp.float32),
        ],
    )(x, W_in_m2, bin2, codebook, c2row, W_out, bout2)

    q = idx.reshape(1, B, T)
    return out, q, loss.reshape(())


# speedup vs baseline: 1.1438x; 1.0009x over previous
import functools

import jax
import jax.numpy as jnp
from jax.experimental import pallas as pl


_COMMIT = 0.25
_KT = 512
_BIG = 1e9


def _vq_body(x_ref, win_ref, bin_ref, cb_ref, c2_ref, wout_ref, bout_ref,
             out_ref, idx_ref, loss_ref, *, n_b, commit_scale):
    b = pl.program_id(0)
    T = x_ref.shape[2]
    CD = cb_ref.shape[1]
    K = cb_ref.shape[0]

    xb = x_ref[0]
    z = jnp.dot(win_ref[...], xb, preferred_element_type=jnp.float32)
    z = z + bin_ref[...]
    zt = z.T
    ztm2 = zt * -2.0

    iota_f = jax.lax.broadcasted_iota(
        jnp.int32, (T, _KT), 1).astype(jnp.float32)

    run_min = jnp.full((T, 1), _BIG, dtype=jnp.float32)
    run_idx = jnp.zeros((T, 1), dtype=jnp.float32)
    run_zq = jnp.zeros((T, CD), dtype=jnp.float32)

    dn = (((1,), (1,)), ((), ()))

    for j in range(K // _KT):
        lo = j * _KT
        cb_tile = cb_ref[lo:lo + _KT, :]
        scores = jax.lax.dot_general(
            ztm2, cb_tile, dn,
            preferred_element_type=jnp.float32)
        dist = scores + c2_ref[:, lo:lo + _KT]
        tmin = jnp.min(dist, axis=1, keepdims=True)
        upd = tmin < run_min
        nm = jnp.where(upd, tmin, run_min)
        key = jnp.where(dist == nm, iota_f, _BIG)
        larg = jnp.min(key, axis=1, keepdims=True)
        onehot = (iota_f == larg).astype(jnp.float32)
        tzq = jnp.dot(onehot, cb_tile,
                      preferred_element_type=jnp.float32)
        run_min = nm
        run_idx = jnp.where(upd, larg + jnp.float32(lo), run_idx)
        run_zq = jnp.where(upd, tzq, run_zq)

    outb = jnp.dot(wout_ref[...], run_zq.T,
                   preferred_element_type=jnp.float32) + bout_ref[...]
    out_ref[0] = outb
    idx_ref[...] = run_idx.astype(jnp.int32)

    diff = run_zq - zt
    part = jnp.sum(diff * diff, axis=(0, 1), keepdims=True)

    @pl.when(b == 0)
    def _init():
        loss_ref[...] = part

    @pl.when(b > 0)
    def _acc():
        loss_ref[...] += part

    @pl.when(b == n_b - 1)
    def _scale():
        loss_ref[...] = loss_ref[...] * commit_scale


@jax.jit
def kernel(x, W_in, b_in, codebook, W_out, b_out):
    B, D, T = x.shape
    K, CD = codebook.shape
    M = B * T

    c2row = jnp.sum(codebook * codebook, axis=1)[None, :]
    bin2 = b_in.reshape(CD, 1)
    bout2 = b_out.reshape(D, 1)

    TS = 1
    TB = T // TS
    n_steps = B * TS
    body = functools.partial(_vq_body, n_b=n_steps,
                             commit_scale=_COMMIT / (M * CD))

    out, idx, loss = pl.pallas_call(
        body,
        grid=(n_steps,),
        in_specs=[
            pl.BlockSpec((1, D, TB), lambda s: (s // TS, 0, s % TS)),
            pl.BlockSpec((CD, D), lambda s: (0, 0)),
            pl.BlockSpec((CD, 1), lambda s: (0, 0)),
            pl.BlockSpec((K, CD), lambda s: (0, 0)),
            pl.BlockSpec((1, K), lambda s: (0, 0)),
            pl.BlockSpec((D, CD), lambda s: (0, 0)),
            pl.BlockSpec((D, 1), lambda s: (0, 0)),
        ],
        out_specs=[
            pl.BlockSpec((1, D, TB), lambda s: (s // TS, 0, s % TS)),
            pl.BlockSpec((TB, 1), lambda s: (s, 0)),
            pl.BlockSpec((1, 1), lambda s: (0, 0)),
        ],
        out_shape=[
            jax.ShapeDtypeStruct((B, D, T), jnp.float32),
            jax.ShapeDtypeStruct((M, 1), jnp.int32),
            jax.ShapeDtypeStruct((1, 1), jnp.float32),
        ],
    )(x, W_in, bin2, codebook, c2row, W_out, bout2)

    q = idx.reshape(1, B, T)
    return out, q, loss.reshape(())
